# baseline (speedup 1.0000x reference)
import functools

import jax
import jax.numpy as jnp
from jax import lax
from jax.experimental import pallas as pl
from jax.experimental.pallas import tpu as pltpu
from jax.experimental.pallas import tpu_sc as plsc

BATCH = 16384
DIM = 32
LANES = 16
NUM_CORES = 2
NUM_SUBCORES = 16
NUM_WORKERS = NUM_CORES * NUM_SUBCORES
BPW = BATCH // NUM_WORKERS
CHUNKS = BPW // LANES
NUM_GENRES = 32
NUM_YEARS = 120
EPS2 = 1e-16


def _rsqrt(x):
    x = jnp.maximum(x, EPS2)
    i = plsc.bitcast(x, jnp.int32)
    y = plsc.bitcast(jnp.int32(0x5F3759DF) - (i >> 1), jnp.float32)
    xh = x * 0.5
    for _ in range(3):
        y = y * (1.5 - xh * y * y)
    return y


def _body(uidx_hbm, midx_hbm, gidx_hbm, yidx_hbm,
          uemb_hbm, memb_hbm, ubias_hbm, mbias_hbm, gemb_hbm, yemb_hbm,
          out_hbm,
          uidx_v, midx_v, gidx_v, yidx_v,
          urows_v, mrows_v, ubv_v, mbv_v, gtab_v, ytab_v, out_v, sem):
    wid = lax.axis_index("s") * NUM_CORES + lax.axis_index("c")
    base = wid * BPW

    pltpu.sync_copy(uidx_hbm.at[pl.ds(base, BPW)], uidx_v)
    pltpu.sync_copy(midx_hbm.at[pl.ds(base, BPW)], midx_v)
    pltpu.sync_copy(gidx_hbm.at[pl.ds(base, BPW)], gidx_v)
    pltpu.sync_copy(yidx_hbm.at[pl.ds(base, BPW)], yidx_v)

    copies = [
        pltpu.async_copy(uemb_hbm.at[uidx_v], urows_v, sem),
        pltpu.async_copy(memb_hbm.at[midx_v], mrows_v, sem),
        pltpu.async_copy(ubias_hbm.at[uidx_v], ubv_v, sem),
        pltpu.async_copy(mbias_hbm.at[midx_v], mbv_v, sem),
        pltpu.async_copy(gemb_hbm, gtab_v, sem),
        pltpu.async_copy(yemb_hbm, ytab_v, sem),
    ]
    for cp in copies:
        cp.wait()

    def chunk(c, carry):
        off = c * LANES
        e16 = jnp.full((LANES,), off, jnp.int32) + lax.iota(jnp.int32, LANES)
        gbase = gidx_v[pl.ds(off, LANES)] << 5
        ybase = yidx_v[pl.ds(off, LANES)] << 5
        zero = jnp.zeros((LANES,), jnp.float32)
        uu = zero; mm = zero; um = zero
        gg = zero; ug = zero
        yy = zero; uy = zero
        for d in range(DIM):
            dd = jnp.full((LANES,), d, jnp.int32)
            u = plsc.load_gather(urows_v, [e16, dd])
            m = plsc.load_gather(mrows_v, [e16, dd])
            g = plsc.load_gather(gtab_v, [gbase + d])
            y = plsc.load_gather(ytab_v, [ybase + d])
            uu = uu + u * u
            mm = mm + m * m
            um = um + u * m
            gg = gg + g * g
            ug = ug + u * g
            yy = yy + y * y
            uy = uy + u * y
        ru = _rsqrt(uu)
        cos_um = um * ru * _rsqrt(mm)
        cos_ug = ug * ru * _rsqrt(gg)
        cos_uy = uy * ru * _rsqrt(yy)
        ub = ubv_v[pl.ds(off, LANES)]
        mb = mbv_v[pl.ds(off, LANES)]
        out_v[pl.ds(off, LANES)] = cos_um * 2.5 + 2.75 + ub + mb + cos_ug + cos_uy
        return carry

    lax.fori_loop(0, CHUNKS, chunk, 0)
    pltpu.sync_copy(out_v, out_hbm.at[pl.ds(base, BPW)])


@jax.jit
def kernel(user_idx, movie_idx, genre_idx, year_idx,
           user_embeds, movie_embeds, user_biases, movie_biases,
           genre_embeds, year_embeds):
    mesh = plsc.VectorSubcoreMesh(core_axis_name="c", subcore_axis_name="s")
    run = functools.partial(
        pl.kernel,
        out_type=jax.ShapeDtypeStruct((BATCH,), jnp.float32),
        mesh=mesh,
        scratch_types=[
            pltpu.VMEM((BPW,), jnp.int32),
            pltpu.VMEM((BPW,), jnp.int32),
            pltpu.VMEM((BPW,), jnp.int32),
            pltpu.VMEM((BPW,), jnp.int32),
            pltpu.VMEM((BPW, DIM), jnp.float32),
            pltpu.VMEM((BPW, DIM), jnp.float32),
            pltpu.VMEM((BPW,), jnp.float32),
            pltpu.VMEM((BPW,), jnp.float32),
            pltpu.VMEM((NUM_GENRES * DIM,), jnp.float32),
            pltpu.VMEM((NUM_YEARS * DIM,), jnp.float32),
            pltpu.VMEM((BPW,), jnp.float32),
            pltpu.SemaphoreType.DMA,
        ],
        compiler_params=pltpu.CompilerParams(
            needs_layout_passes=False, use_tc_tiling_on_sc=False),
    )(_body)
    scale = jnp.float32(1.0000001)
    return run(user_idx.astype(jnp.int32), movie_idx.astype(jnp.int32),
               genre_idx.astype(jnp.int32), year_idx.astype(jnp.int32),
               user_embeds * scale, movie_embeds * scale,
               user_biases.reshape(-1), movie_biases.reshape(-1),
               genre_embeds.reshape(-1), year_embeds.reshape(-1))


# speedup vs baseline: 1.5948x; 1.5948x over previous
import functools

import jax
import jax.numpy as jnp
from jax import lax
from jax.experimental import pallas as pl
from jax.experimental.pallas import tpu as pltpu
from jax.experimental.pallas import tpu_sc as plsc

BATCH = 16384
DIM = 32
LANES = 16
NUM_CORES = 2
NUM_SUBCORES = 16
NUM_WORKERS = NUM_CORES * NUM_SUBCORES
BPW = BATCH // NUM_WORKERS
CHUNKS = BPW // LANES
NUM_GENRES = 32
NUM_YEARS = 120
EPS2 = 1e-16


def _rsqrt(x):
    x = jnp.maximum(x, EPS2)
    i = plsc.bitcast(x, jnp.int32)
    y = plsc.bitcast(jnp.int32(0x5F3759DF) - (i >> 1), jnp.float32)
    xh = x * 0.5
    for _ in range(3):
        y = y * (1.5 - xh * y * y)
    return y


def _body(uidx_hbm, midx_hbm, gidx_hbm, yidx_hbm,
          uemb_hbm, memb_hbm, ubias_hbm, mbias_hbm, gemb_hbm, yemb_hbm,
          out_hbm,
          uidx_v, midx_v, gidx_v, yidx_v,
          urows_v, mrows_v, ubv_v, mbv_v, gtab_v, ytab_v, out_v, sem):
    wid = lax.axis_index("s") * NUM_CORES + lax.axis_index("c")
    base = wid * BPW

    pltpu.sync_copy(uidx_hbm.at[pl.ds(base, BPW)], uidx_v)
    pltpu.sync_copy(midx_hbm.at[pl.ds(base, BPW)], midx_v)
    pltpu.sync_copy(gidx_hbm.at[pl.ds(base, BPW)], gidx_v)
    pltpu.sync_copy(yidx_hbm.at[pl.ds(base, BPW)], yidx_v)

    copies = [
        pltpu.async_copy(uemb_hbm.at[uidx_v], urows_v, sem),
        pltpu.async_copy(memb_hbm.at[midx_v], mrows_v, sem),
        pltpu.async_copy(ubias_hbm.at[uidx_v], ubv_v, sem),
        pltpu.async_copy(mbias_hbm.at[midx_v], mbv_v, sem),
        pltpu.async_copy(gemb_hbm, gtab_v, sem),
        pltpu.async_copy(yemb_hbm, ytab_v, sem),
    ]
    for cp in copies:
        cp.wait()

    def chunk(c, carry):
        off = c * LANES
        e16 = jnp.full((LANES,), off, jnp.int32) + lax.iota(jnp.int32, LANES)
        gbase = gidx_v[pl.ds(off, LANES)] << 5
        ybase = yidx_v[pl.ds(off, LANES)] << 5
        zero = jnp.zeros((LANES,), jnp.float32)
        uu = zero; mm = zero; um = zero
        gg = zero; ug = zero
        yy = zero; uy = zero
        for d in range(DIM):
            dd = jnp.full((LANES,), d, jnp.int32)
            u = plsc.load_gather(urows_v, [e16, dd])
            m = plsc.load_gather(mrows_v, [e16, dd])
            g = plsc.load_gather(gtab_v, [gbase + d])
            y = plsc.load_gather(ytab_v, [ybase + d])
            uu = uu + u * u
            mm = mm + m * m
            um = um + u * m
            gg = gg + g * g
            ug = ug + u * g
            yy = yy + y * y
            uy = uy + u * y
        ru = _rsqrt(uu)
        cos_um = um * ru * _rsqrt(mm)
        cos_ug = ug * ru * _rsqrt(gg)
        cos_uy = uy * ru * _rsqrt(yy)
        ub = ubv_v[pl.ds(off, LANES)]
        mb = mbv_v[pl.ds(off, LANES)]
        out_v[pl.ds(off, LANES)] = cos_um * 2.5 + 2.75 + ub + mb + cos_ug + cos_uy
        return carry

    lax.fori_loop(0, CHUNKS, chunk, 0)
    pltpu.sync_copy(out_v, out_hbm.at[pl.ds(base, BPW)])


@jax.jit
def kernel(user_idx, movie_idx, genre_idx, year_idx,
           user_embeds, movie_embeds, user_biases, movie_biases,
           genre_embeds, year_embeds):
    mesh = plsc.VectorSubcoreMesh(core_axis_name="c", subcore_axis_name="s")
    run = functools.partial(
        pl.kernel,
        out_type=jax.ShapeDtypeStruct((BATCH,), jnp.float32),
        mesh=mesh,
        scratch_types=[
            pltpu.VMEM((BPW,), jnp.int32),
            pltpu.VMEM((BPW,), jnp.int32),
            pltpu.VMEM((BPW,), jnp.int32),
            pltpu.VMEM((BPW,), jnp.int32),
            pltpu.VMEM((BPW, DIM), jnp.float32),
            pltpu.VMEM((BPW, DIM), jnp.float32),
            pltpu.VMEM((BPW,), jnp.float32),
            pltpu.VMEM((BPW,), jnp.float32),
            pltpu.VMEM((NUM_GENRES * DIM,), jnp.float32),
            pltpu.VMEM((NUM_YEARS * DIM,), jnp.float32),
            pltpu.VMEM((BPW,), jnp.float32),
            pltpu.SemaphoreType.DMA,
        ],
        compiler_params=pltpu.CompilerParams(
            needs_layout_passes=False, use_tc_tiling_on_sc=False),
    )(_body)
    return run(user_idx.astype(jnp.int32), movie_idx.astype(jnp.int32),
               genre_idx.astype(jnp.int32), year_idx.astype(jnp.int32),
               user_embeds, movie_embeds,
               user_biases.reshape(-1), movie_biases.reshape(-1),
               genre_embeds.reshape(-1), year_embeds.reshape(-1))
